# baseline (speedup 1.0000x reference)
import jax
import jax.numpy as jnp
from jax import lax
from jax.experimental import pallas as pl
from jax.experimental.pallas import tpu as pltpu
from jax.experimental.pallas import tpu_sc as plsc

NC = 2
NS = 16
L = 16
NW = NC * NS

B = 16384
D = 64
CHUNK = 128
B_PER_W = B // NW
NCHUNK = B_PER_W // CHUNK
GROUPS_PER_CHUNK = CHUNK // L


def _bpr_body(u_tab, i_tab, uidx_hbm, pidx_hbm, nidx_hbm, out_hbm,
              uidx_v, pidx_v, nidx_v, urows, prows, nrows, out_v,
              isem, gsem):
    wid = lax.axis_index("s") * NC + lax.axis_index("c")
    base = pl.multiple_of(wid * B_PER_W, 8)

    idx_copies = [
        pltpu.async_copy(uidx_hbm.at[pl.ds(base, B_PER_W)], uidx_v, isem),
        pltpu.async_copy(pidx_hbm.at[pl.ds(base, B_PER_W)], pidx_v, isem),
        pltpu.async_copy(nidx_hbm.at[pl.ds(base, B_PER_W)], nidx_v, isem),
    ]
    for c in idx_copies:
        c.wait()

    copies = []
    for j in range(NCHUNK):
        src = pl.ds(j * CHUNK, CHUNK)
        dst = pl.ds(j * CHUNK, CHUNK)
        copies.append(pltpu.async_copy(
            u_tab.at[uidx_v.at[src]], urows.at[dst], gsem))
        copies.append(pltpu.async_copy(
            i_tab.at[pidx_v.at[src]], prows.at[dst], gsem))
        copies.append(pltpu.async_copy(
            i_tab.at[nidx_v.at[src]], nrows.at[dst], gsem))

    iota = lax.iota(jnp.int32, L)

    def group_body(g, carry):
        rb = g * L
        rowids = rb + iota
        accs = [jnp.zeros((L,), jnp.float32) for _ in range(4)]
        for d in range(0, D, 4):
            for k in range(4):
                dv = jnp.full((L,), d + k, jnp.int32)
                u = plsc.load_gather(urows, [rowids, dv])
                p = plsc.load_gather(prows, [rowids, dv])
                n = plsc.load_gather(nrows, [rowids, dv])
                accs[k] = accs[k] + u * (p - n)
        out_v[pl.ds(rb, L)] = (accs[0] + accs[1]) + (accs[2] + accs[3])
        return carry

    for j in range(NCHUNK):
        for c in copies[3 * j:3 * j + 3]:
            c.wait()
        lax.fori_loop(j * GROUPS_PER_CHUNK, (j + 1) * GROUPS_PER_CHUNK,
                      group_body, 0)

    pltpu.sync_copy(out_v, out_hbm.at[pl.ds(base, B_PER_W)])


@jax.jit
def _bpr_sc(user_table, item_table, uidx, pidx, nidx):
    mesh = plsc.VectorSubcoreMesh(
        core_axis_name="c", subcore_axis_name="s", num_cores=NC, num_subcores=NS
    )
    return pl.kernel(
        _bpr_body,
        out_type=jax.ShapeDtypeStruct((B,), jnp.float32),
        mesh=mesh,
        scratch_types=[
            pltpu.VMEM((B_PER_W,), jnp.int32),
            pltpu.VMEM((B_PER_W,), jnp.int32),
            pltpu.VMEM((B_PER_W,), jnp.int32),
            pltpu.VMEM((B_PER_W, D), jnp.float32),
            pltpu.VMEM((B_PER_W, D), jnp.float32),
            pltpu.VMEM((B_PER_W, D), jnp.float32),
            pltpu.VMEM((B_PER_W,), jnp.float32),
            pltpu.SemaphoreType.DMA,
            pltpu.SemaphoreType.DMA,
        ],
        compiler_params=pltpu.CompilerParams(
            needs_layout_passes=False, use_tc_tiling_on_sc=False),
    )(user_table, item_table, uidx, pidx, nidx)


def kernel(user_table, item_table, user_input, pos_item_input, neg_item_input):
    out = _bpr_sc(user_table, item_table,
                  user_input.reshape(-1).astype(jnp.int32),
                  pos_item_input.reshape(-1).astype(jnp.int32),
                  neg_item_input.reshape(-1).astype(jnp.int32))
    return out.reshape(B, 1)


# speedup vs baseline: 1.0129x; 1.0129x over previous
import jax
import jax.numpy as jnp
from jax import lax
from jax.experimental import pallas as pl
from jax.experimental.pallas import tpu as pltpu
from jax.experimental.pallas import tpu_sc as plsc

NC = 2
NS = 16
L = 16
NW = NC * NS

B = 16384
D = 64
CHUNK = 128
B_PER_W = B // NW
NCHUNK = B_PER_W // CHUNK
GROUPS_PER_CHUNK = CHUNK // L

_MESH = dict(core_axis_name="c", subcore_axis_name="s",
             num_cores=NC, num_subcores=NS)
_PARAMS = dict(needs_layout_passes=False, use_tc_tiling_on_sc=False)


def _gather_u_body(u_tab, uidx_hbm, ustage_hbm, uidx_v, urows, isem, gsem):
    wid = lax.axis_index("s") * NC + lax.axis_index("c")
    base = pl.multiple_of(wid * B_PER_W, 8)

    pltpu.async_copy(uidx_hbm.at[pl.ds(base, B_PER_W)], uidx_v, isem).wait()
    copies = []
    for j in range(NCHUNK):
        sl = pl.ds(j * CHUNK, CHUNK)
        copies.append(pltpu.async_copy(
            u_tab.at[uidx_v.at[sl]], urows.at[sl], gsem))
    for c in copies:
        c.wait()
    pltpu.sync_copy(urows, ustage_hbm.at[pl.ds(base, B_PER_W), :])


def _bpr_main_body(i_tab, ustage_hbm, pidx_hbm, nidx_hbm, out_hbm,
                   pidx_v, nidx_v, urows, prows, nrows, out_v, isem, gsem):
    wid = lax.axis_index("s") * NC + lax.axis_index("c")
    base = pl.multiple_of(wid * B_PER_W, 8)

    idx_copies = [
        pltpu.async_copy(pidx_hbm.at[pl.ds(base, B_PER_W)], pidx_v, isem),
        pltpu.async_copy(nidx_hbm.at[pl.ds(base, B_PER_W)], nidx_v, isem),
    ]
    for c in idx_copies:
        c.wait()

    copies = []
    for j in range(NCHUNK):
        sl = pl.ds(j * CHUNK, CHUNK)
        gsl = pl.ds(base + j * CHUNK, CHUNK)
        copies.append(pltpu.async_copy(
            ustage_hbm.at[gsl, :], urows.at[sl], gsem))
        copies.append(pltpu.async_copy(
            i_tab.at[pidx_v.at[sl]], prows.at[sl], gsem))
        copies.append(pltpu.async_copy(
            i_tab.at[nidx_v.at[sl]], nrows.at[sl], gsem))

    iota = lax.iota(jnp.int32, L)

    def group_body(g, carry):
        rb = g * L
        rowids = rb + iota
        acc0 = jnp.zeros((L,), jnp.float32)
        acc1 = jnp.zeros((L,), jnp.float32)
        for d in range(0, D, 2):
            dv0 = jnp.full((L,), d, jnp.int32)
            dv1 = jnp.full((L,), d + 1, jnp.int32)
            u0 = plsc.load_gather(urows, [rowids, dv0])
            p0 = plsc.load_gather(prows, [rowids, dv0])
            n0 = plsc.load_gather(nrows, [rowids, dv0])
            u1 = plsc.load_gather(urows, [rowids, dv1])
            p1 = plsc.load_gather(prows, [rowids, dv1])
            n1 = plsc.load_gather(nrows, [rowids, dv1])
            acc0 = acc0 + u0 * (p0 - n0)
            acc1 = acc1 + u1 * (p1 - n1)
        out_v[pl.ds(rb, L)] = acc0 + acc1
        return carry

    for j in range(NCHUNK):
        for c in copies[3 * j:3 * j + 3]:
            c.wait()
        lax.fori_loop(j * GROUPS_PER_CHUNK, (j + 1) * GROUPS_PER_CHUNK,
                      group_body, 0)

    pltpu.sync_copy(out_v, out_hbm.at[pl.ds(base, B_PER_W)])


@jax.jit
def _bpr_sc(user_table, item_table, uidx, pidx, nidx):
    ustage = pl.kernel(
        _gather_u_body,
        out_type=jax.ShapeDtypeStruct((B, D), jnp.float32),
        mesh=plsc.VectorSubcoreMesh(**_MESH),
        scratch_types=[
            pltpu.VMEM((B_PER_W,), jnp.int32),
            pltpu.VMEM((B_PER_W, D), jnp.float32),
            pltpu.SemaphoreType.DMA,
            pltpu.SemaphoreType.DMA,
        ],
        compiler_params=pltpu.CompilerParams(**_PARAMS),
    )(user_table, uidx)

    return pl.kernel(
        _bpr_main_body,
        out_type=jax.ShapeDtypeStruct((B,), jnp.float32),
        mesh=plsc.VectorSubcoreMesh(**_MESH),
        scratch_types=[
            pltpu.VMEM((B_PER_W,), jnp.int32),
            pltpu.VMEM((B_PER_W,), jnp.int32),
            pltpu.VMEM((B_PER_W, D), jnp.float32),
            pltpu.VMEM((B_PER_W, D), jnp.float32),
            pltpu.VMEM((B_PER_W, D), jnp.float32),
            pltpu.VMEM((B_PER_W,), jnp.float32),
            pltpu.SemaphoreType.DMA,
            pltpu.SemaphoreType.DMA,
        ],
        compiler_params=pltpu.CompilerParams(**_PARAMS),
    )(item_table, ustage, pidx, nidx)


def kernel(user_table, item_table, user_input, pos_item_input, neg_item_input):
    out = _bpr_sc(user_table, item_table,
                  user_input.reshape(-1).astype(jnp.int32),
                  pos_item_input.reshape(-1).astype(jnp.int32),
                  neg_item_input.reshape(-1).astype(jnp.int32))
    return out.reshape(B, 1)


# speedup vs baseline: 1.0229x; 1.0099x over previous
import jax
import jax.numpy as jnp
from jax import lax
from jax.experimental import pallas as pl
from jax.experimental.pallas import tpu as pltpu
from jax.experimental.pallas import tpu_sc as plsc

NC = 2
NS = 16
L = 16
NW = NC * NS

B = 16384
D = 64
CHUNK = 128
B_PER_W = B // NW
NCHUNK = B_PER_W // CHUNK
GROUPS_PER_CHUNK = CHUNK // L


def _bpr_body(u_tab, i_tab, uidx_hbm, pidx_hbm, nidx_hbm, out_hbm,
              uidx_v, pidx_v, nidx_v, urows, prows, nrows, out_v,
              isem, gsem):
    wid = lax.axis_index("s") * NC + lax.axis_index("c")
    base = pl.multiple_of(wid * B_PER_W, 8)

    idx_copies = [
        pltpu.async_copy(uidx_hbm.at[pl.ds(base, B_PER_W)], uidx_v, isem),
        pltpu.async_copy(pidx_hbm.at[pl.ds(base, B_PER_W)], pidx_v, isem),
        pltpu.async_copy(nidx_hbm.at[pl.ds(base, B_PER_W)], nidx_v, isem),
    ]
    for c in idx_copies:
        c.wait()

    copies = []
    for j in range(NCHUNK):
        src = pl.ds(j * CHUNK, CHUNK)
        dst = pl.ds(j * CHUNK, CHUNK)
        copies.append(pltpu.async_copy(
            u_tab.at[uidx_v.at[src]], urows.at[dst], gsem))
        copies.append(pltpu.async_copy(
            i_tab.at[pidx_v.at[src]], prows.at[dst], gsem))
        copies.append(pltpu.async_copy(
            i_tab.at[nidx_v.at[src]], nrows.at[dst], gsem))

    iota = lax.iota(jnp.int32, L)

    def group_body(g, carry):
        rb = g * L
        rowids = rb + iota
        acc0 = jnp.zeros((L,), jnp.float32)
        acc1 = jnp.zeros((L,), jnp.float32)
        for d in range(0, D, 2):
            dv0 = jnp.full((L,), d, jnp.int32)
            dv1 = jnp.full((L,), d + 1, jnp.int32)
            u0 = plsc.load_gather(urows, [rowids, dv0])
            p0 = plsc.load_gather(prows, [rowids, dv0])
            n0 = plsc.load_gather(nrows, [rowids, dv0])
            u1 = plsc.load_gather(urows, [rowids, dv1])
            p1 = plsc.load_gather(prows, [rowids, dv1])
            n1 = plsc.load_gather(nrows, [rowids, dv1])
            acc0 = acc0 + u0 * (p0 - n0)
            acc1 = acc1 + u1 * (p1 - n1)
        out_v[pl.ds(rb, L)] = acc0 + acc1
        return carry

    for j in range(NCHUNK):
        for c in copies[3 * j:3 * j + 3]:
            c.wait()
        lax.fori_loop(j * GROUPS_PER_CHUNK, (j + 1) * GROUPS_PER_CHUNK,
                      group_body, 0)

    pltpu.sync_copy(out_v, out_hbm.at[pl.ds(base, B_PER_W)])


@jax.jit
def _bpr_sc(user_table, item_table, uidx, pidx, nidx):
    mesh = plsc.VectorSubcoreMesh(
        core_axis_name="c", subcore_axis_name="s", num_cores=NC, num_subcores=NS
    )
    return pl.kernel(
        _bpr_body,
        out_type=jax.ShapeDtypeStruct((B,), jnp.float32),
        mesh=mesh,
        scratch_types=[
            pltpu.VMEM((B_PER_W,), jnp.int32),
            pltpu.VMEM((B_PER_W,), jnp.int32),
            pltpu.VMEM((B_PER_W,), jnp.int32),
            pltpu.VMEM((B_PER_W, D), jnp.float32),
            pltpu.VMEM((B_PER_W, D), jnp.float32),
            pltpu.VMEM((B_PER_W, D), jnp.float32),
            pltpu.VMEM((B_PER_W,), jnp.float32),
            pltpu.SemaphoreType.DMA,
            pltpu.SemaphoreType.DMA,
        ],
        compiler_params=pltpu.CompilerParams(
            needs_layout_passes=False, use_tc_tiling_on_sc=False),
    )(user_table, item_table, uidx, pidx, nidx)


def kernel(user_table, item_table, user_input, pos_item_input, neg_item_input):
    out = _bpr_sc(user_table, item_table,
                  user_input.reshape(-1).astype(jnp.int32),
                  pos_item_input.reshape(-1).astype(jnp.int32),
                  neg_item_input.reshape(-1).astype(jnp.int32))
    return out.reshape(B, 1)
